# baseline (speedup 1.0000x reference)
import functools

import jax
import jax.numpy as jnp
from jax import lax
from jax.experimental import pallas as pl
from jax.experimental.pallas import tpu as pltpu
from jax.experimental.pallas import tpu_sc as plsc

NC, NS = 2, 16
NW = NC * NS
LANES = 16


def _deg_partials(dst, w, N):
    E = dst.shape[0]
    EW = E // NW
    CH = 400
    NCH = N // CH
    mesh = plsc.VectorSubcoreMesh(core_axis_name="c", subcore_axis_name="s",
                                  num_cores=NC, num_subcores=NS)

    @functools.partial(
        pl.kernel,
        out_type=jax.ShapeDtypeStruct((NC * N,), jnp.float32),
        mesh=mesh,
        scratch_types=[
            pltpu.VMEM((EW,), jnp.int32),
            pltpu.VMEM((EW,), jnp.float32),
            pltpu.VMEM((CH,), jnp.float32),
            pltpu.VMEM_SHARED((N,), jnp.float32),
        ],
    )
    def k(dst_hbm, w_hbm, out_hbm, dst_v, w_v, zbuf, deg_sp):
        c = lax.axis_index("c")
        s = lax.axis_index("s")
        wid = c * NS + s

        def zb(i, carry):
            zbuf[pl.ds(i * LANES, LANES)] = jnp.zeros((LANES,), jnp.float32)
            return carry

        lax.fori_loop(0, CH // LANES, zb, 0)
        for j in range((NCH + NS - 1) // NS):
            ch = s + j * NS

            @pl.when(ch < NCH)
            def _():
                pltpu.sync_copy(zbuf, deg_sp.at[pl.ds(ch * CH, CH)])

        plsc.subcore_barrier()
        base = wid * EW
        pltpu.sync_copy(dst_hbm.at[pl.ds(base, EW)], dst_v)
        pltpu.sync_copy(w_hbm.at[pl.ds(base, EW)], w_v)
        pltpu.sync_copy(w_v, deg_sp.at[dst_v], add=True)
        plsc.subcore_barrier()
        for j in range((NCH + NS - 1) // NS):
            ch = s + j * NS

            @pl.when(ch < NCH)
            def _():
                pltpu.sync_copy(deg_sp.at[pl.ds(ch * CH, CH)], zbuf)
                pltpu.sync_copy(zbuf, out_hbm.at[pl.ds(c * N + ch * CH, CH)])

    return k(dst, w).reshape(NC, N)


def _aggregate(hp, src, dst, w):
    N, D = hp.shape
    E = src.shape[0]
    EW = E // NW
    B = 128
    NWIN = EW // B
    RCH = 80
    NRCH = N // RCH
    KD = D // LANES
    mesh = plsc.VectorSubcoreMesh(core_axis_name="c", subcore_axis_name="s",
                                  num_cores=NC, num_subcores=NS)

    @functools.partial(
        pl.kernel,
        out_type=jax.ShapeDtypeStruct((NC, N, D), jnp.float32),
        mesh=mesh,
        scratch_types=[
            pltpu.VMEM((EW,), jnp.int32),
            [pltpu.VMEM((B,), jnp.int32) for _ in range(2)],
            [pltpu.VMEM((B,), jnp.float32) for _ in range(2)],
            [pltpu.VMEM((B, D), jnp.float32) for _ in range(2)],
            pltpu.VMEM_SHARED((N, D), jnp.float32),
            [pltpu.SemaphoreType.DMA for _ in range(2)],
            [pltpu.SemaphoreType.DMA for _ in range(2)],
            [pltpu.SemaphoreType.DMA for _ in range(2)],
        ],
    )
    def k(h_hbm, src_hbm, dst_hbm, w_hbm, out_hbm,
          src_a, dst_v, w_v, rows_v, acc, gsem, ssem, dsem):
        c = lax.axis_index("c")
        s = lax.axis_index("s")
        wid = c * NS + s
        tbase = wid * EW

        pltpu.sync_copy(src_hbm.at[pl.ds(tbase, EW)], src_a)

        def zr(e, carry):
            for kk in range(KD):
                rows_v[0][e, pl.ds(kk * LANES, LANES)] = jnp.zeros(
                    (LANES,), jnp.float32)
            return carry

        lax.fori_loop(0, RCH, zr, 0)
        for j in range((NRCH + NS - 1) // NS):
            ch = s + j * NS

            @pl.when(ch < NRCH)
            def _():
                pltpu.sync_copy(rows_v[0].at[pl.ds(0, RCH)],
                                acc.at[pl.ds(ch * RCH, RCH)])

        plsc.subcore_barrier()

        def issue_gather(p, win):
            return pltpu.async_copy(h_hbm.at[src_a.at[pl.ds(win * B, B)]],
                                    rows_v[p], gsem[p])

        def wait_gather(p, win):
            pltpu.make_async_copy(h_hbm.at[src_a.at[pl.ds(win * B, B)]],
                                  rows_v[p], gsem[p]).wait()

        def issue_dstload(p, win):
            pltpu.async_copy(dst_hbm.at[pl.ds(tbase + win * B, B)],
                             dst_v[p], dsem[p])
            pltpu.async_copy(w_hbm.at[pl.ds(tbase + win * B, B)],
                             w_v[p], dsem[p])

        def wait_dstload(p, win):
            pltpu.make_async_copy(dst_hbm.at[pl.ds(tbase + win * B, B)],
                                  dst_v[p], dsem[p]).wait()
            pltpu.make_async_copy(w_hbm.at[pl.ds(tbase + win * B, B)],
                                  w_v[p], dsem[p]).wait()

        UNROLL = 4

        def scale(p, win):
            def grp(g, carry):
                for u in range(UNROLL):
                    gg = g * UNROLL + u
                    w16 = w_v[p][pl.ds(gg * LANES, LANES)]
                    for j in range(LANES):
                        we = w16[j]
                        e = gg * LANES + j
                        for kk in range(KD):
                            sl = pl.ds(kk * LANES, LANES)
                            rows_v[p][e, sl] = rows_v[p][e, sl] * we
                return carry

            lax.fori_loop(0, B // LANES // UNROLL, grp, 0)

        def issue_scatter(p, win):
            return pltpu.async_copy(rows_v[p], acc.at[dst_v[p]],
                                    ssem[p], add=True)

        def wait_scatter(p, win):
            pltpu.make_async_copy(rows_v[p], acc.at[dst_v[p]],
                                  ssem[p]).wait()

        issue_dstload(0, 0)
        issue_gather(0, 0)

        def step(win, p, first, last):
            wait_gather(p, win)
            if not last:
                if not first:
                    wait_scatter(1 - p, win - 1)
                issue_gather(1 - p, win + 1)
                issue_dstload(1 - p, win + 1)
            scale(p, win)
            wait_dstload(p, win)
            issue_scatter(p, win)
            if last:
                wait_scatter(1 - p, win - 1)
                wait_scatter(p, win)

        step(0, 0, True, False)
        npairs = (NWIN - 2) // 2

        def pair(q, carry):
            win = 1 + 2 * q
            step(win, 1, False, False)
            step(win + 1, 0, False, False)
            return carry

        lax.fori_loop(0, npairs, pair, 0)
        if NWIN - 1 - 2 * npairs == 2:
            step(NWIN - 2, (NWIN - 2) % 2, False, False)
        step(NWIN - 1, (NWIN - 1) % 2, False, True)
        plsc.subcore_barrier()
        for j in range((NRCH + NS - 1) // NS):
            ch = s + j * NS

            @pl.when(ch < NRCH)
            def _():
                pltpu.sync_copy(acc.at[pl.ds(ch * RCH, RCH)],
                                rows_v[0].at[pl.ds(0, RCH)])
                pltpu.sync_copy(rows_v[0].at[pl.ds(0, RCH)],
                                out_hbm.at[c, pl.ds(ch * RCH, RCH)])

    return k(hp, src, dst, w)


_RB = 2000


def _first_layer_h(x, W, degp):
    N, D = x.shape

    def body(x_ref, w_ref, dp_ref, o_ref, d_ref):
        deg = dp_ref[:, 0] + dp_ref[:, 1] + 1.0
        dis = jnp.where(deg > 0, lax.rsqrt(deg), 0.0)[:, None]
        d_ref[...] = dis
        o_ref[...] = dis * jnp.dot(
            x_ref[...], w_ref[...], preferred_element_type=jnp.float32)

    return pl.pallas_call(
        body,
        grid=(N // _RB,),
        in_specs=[
            pl.BlockSpec((_RB, D), lambda i: (i, 0)),
            pl.BlockSpec((D, D), lambda i: (0, 0)),
            pl.BlockSpec((_RB, 2), lambda i: (i, 0)),
        ],
        out_specs=[
            pl.BlockSpec((_RB, D), lambda i: (i, 0)),
            pl.BlockSpec((_RB, 1), lambda i: (i, 0)),
        ],
        out_shape=[
            jax.ShapeDtypeStruct((N, D), jnp.float32),
            jax.ShapeDtypeStruct((N, 1), jnp.float32),
        ],
    )(x, W, degp)


def _mid_layer(accp, hp, dis, b, W):
    N, D = hp.shape

    def body(a_ref, h_ref, d_ref, b_ref, w_ref, o_ref):
        ssum = a_ref[0] + a_ref[1] + h_ref[...]
        x2 = jnp.maximum(d_ref[...] * ssum + b_ref[...], 0.0)
        o_ref[...] = d_ref[...] * jnp.dot(
            x2, w_ref[...], preferred_element_type=jnp.float32)

    return pl.pallas_call(
        body,
        grid=(N // _RB,),
        in_specs=[
            pl.BlockSpec((NC, _RB, D), lambda i: (0, i, 0)),
            pl.BlockSpec((_RB, D), lambda i: (i, 0)),
            pl.BlockSpec((_RB, 1), lambda i: (i, 0)),
            pl.BlockSpec((1, D), lambda i: (0, 0)),
            pl.BlockSpec((D, D), lambda i: (0, 0)),
        ],
        out_specs=pl.BlockSpec((_RB, D), lambda i: (i, 0)),
        out_shape=jax.ShapeDtypeStruct((N, D), jnp.float32),
    )(accp, hp, dis, b, W)


def _final_layer(accp, hp, dis, b, Wl, bl):
    N, D = hp.shape

    def body(a_ref, h_ref, d_ref, b_ref, wl_ref, bl_ref, o_ref):
        ssum = a_ref[0] + a_ref[1] + h_ref[...]
        x3 = jnp.maximum(d_ref[...] * ssum + b_ref[...], 0.0)
        o_ref[...] = jnp.dot(
            x3, wl_ref[...], preferred_element_type=jnp.float32) + bl_ref[...]

    return pl.pallas_call(
        body,
        grid=(N // _RB,),
        in_specs=[
            pl.BlockSpec((NC, _RB, D), lambda i: (0, i, 0)),
            pl.BlockSpec((_RB, D), lambda i: (i, 0)),
            pl.BlockSpec((_RB, 1), lambda i: (i, 0)),
            pl.BlockSpec((1, D), lambda i: (0, 0)),
            pl.BlockSpec((D, 1), lambda i: (0, 0)),
            pl.BlockSpec((1, 1), lambda i: (0, 0)),
        ],
        out_specs=pl.BlockSpec((_RB, 1), lambda i: (i, 0)),
        out_shape=jax.ShapeDtypeStruct((N, 1), jnp.float32),
    )(accp, hp, dis, b, Wl, bl)


def kernel(w_x, edge_index, edge_weight, W1, b1, W2, b2, Wl, bl):
    N = w_x.shape[0]
    src = edge_index[0].astype(jnp.int32)
    dst = edge_index[1].astype(jnp.int32)
    w = edge_weight.astype(jnp.float32)

    E = src.shape[0]
    EP = ((E // NW + 127) // 128) * 128 * NW
    npad = EP - E
    if npad:
        pad_idx = (jnp.arange(npad, dtype=jnp.int32) * 97) % N
        src_p = jnp.concatenate([src, pad_idx])
        dst_p = jnp.concatenate([dst, pad_idx])
        w_p = jnp.concatenate([w, jnp.zeros((npad,), jnp.float32)])
    else:
        src_p, dst_p, w_p = src, dst, w

    degp = _deg_partials(dst, w, N)
    h1p, dis = _first_layer_h(w_x, W1, degp.T)
    acc1 = _aggregate(h1p, src_p, dst_p, w_p)
    h2p = _mid_layer(acc1, h1p, dis, b1.reshape(1, -1), W2)
    acc2 = _aggregate(h2p, src_p, dst_p, w_p)
    return _final_layer(acc2, h2p, dis, b2.reshape(1, -1), Wl,
                        bl.reshape(1, 1))


# speedup vs baseline: 1.0182x; 1.0182x over previous
import functools

import jax
import jax.numpy as jnp
from jax import lax
from jax.experimental import pallas as pl
from jax.experimental.pallas import tpu as pltpu
from jax.experimental.pallas import tpu_sc as plsc

NC, NS = 2, 16
NW = NC * NS
LANES = 16


def _deg_partials(dst, w, N):
    E = dst.shape[0]
    EW = E // NW
    CH = 400
    NCH = N // CH
    mesh = plsc.VectorSubcoreMesh(core_axis_name="c", subcore_axis_name="s",
                                  num_cores=NC, num_subcores=NS)

    @functools.partial(
        pl.kernel,
        out_type=jax.ShapeDtypeStruct((NC * N,), jnp.float32),
        mesh=mesh,
        scratch_types=[
            pltpu.VMEM((EW,), jnp.int32),
            pltpu.VMEM((EW,), jnp.float32),
            pltpu.VMEM((CH,), jnp.float32),
            pltpu.VMEM_SHARED((N,), jnp.float32),
        ],
    )
    def k(dst_hbm, w_hbm, out_hbm, dst_v, w_v, zbuf, deg_sp):
        c = lax.axis_index("c")
        s = lax.axis_index("s")
        wid = c * NS + s

        def zb(i, carry):
            zbuf[pl.ds(i * LANES, LANES)] = jnp.zeros((LANES,), jnp.float32)
            return carry

        lax.fori_loop(0, CH // LANES, zb, 0)
        for j in range((NCH + NS - 1) // NS):
            ch = s + j * NS

            @pl.when(ch < NCH)
            def _():
                pltpu.sync_copy(zbuf, deg_sp.at[pl.ds(ch * CH, CH)])

        plsc.subcore_barrier()
        base = wid * EW
        pltpu.sync_copy(dst_hbm.at[pl.ds(base, EW)], dst_v)
        pltpu.sync_copy(w_hbm.at[pl.ds(base, EW)], w_v)
        pltpu.sync_copy(w_v, deg_sp.at[dst_v], add=True)
        plsc.subcore_barrier()
        for j in range((NCH + NS - 1) // NS):
            ch = s + j * NS

            @pl.when(ch < NCH)
            def _():
                pltpu.sync_copy(deg_sp.at[pl.ds(ch * CH, CH)], zbuf)
                pltpu.sync_copy(zbuf, out_hbm.at[pl.ds(c * N + ch * CH, CH)])

    return k(dst, w).reshape(NC, N)


def _aggregate(hp, src, dst, w):
    N, D = hp.shape
    E = src.shape[0]
    EW = E // NW
    B = 128
    NWIN = EW // B
    RCH = 80
    NRCH = N // RCH
    KD = D // LANES
    mesh = plsc.VectorSubcoreMesh(core_axis_name="c", subcore_axis_name="s",
                                  num_cores=NC, num_subcores=NS)

    @functools.partial(
        pl.kernel,
        out_type=jax.ShapeDtypeStruct((NC, N, D), jnp.float32),
        mesh=mesh,
        scratch_types=[
            pltpu.VMEM((EW,), jnp.int32),
            [pltpu.VMEM((B,), jnp.int32) for _ in range(2)],
            [pltpu.VMEM((B,), jnp.float32) for _ in range(2)],
            [pltpu.VMEM((B, D), jnp.float32) for _ in range(2)],
            pltpu.VMEM_SHARED((N, D), jnp.float32),
            [pltpu.SemaphoreType.DMA for _ in range(2)],
            [pltpu.SemaphoreType.DMA for _ in range(2)],
            [pltpu.SemaphoreType.DMA for _ in range(2)],
        ],
    )
    def k(h_hbm, src_hbm, dst_hbm, w_hbm, out_hbm,
          src_a, dst_v, w_v, rows_v, acc, gsem, ssem, dsem):
        c = lax.axis_index("c")
        s = lax.axis_index("s")
        wid = c * NS + s
        tbase = wid * EW

        pltpu.sync_copy(src_hbm.at[pl.ds(tbase, EW)], src_a)

        def zr(e, carry):
            for kk in range(KD):
                rows_v[0][e, pl.ds(kk * LANES, LANES)] = jnp.zeros(
                    (LANES,), jnp.float32)
            return carry

        lax.fori_loop(0, RCH, zr, 0)
        for j in range((NRCH + NS - 1) // NS):
            ch = s + j * NS

            @pl.when(ch < NRCH)
            def _():
                pltpu.sync_copy(rows_v[0].at[pl.ds(0, RCH)],
                                acc.at[pl.ds(ch * RCH, RCH)])

        plsc.subcore_barrier()

        def issue_gather(p, win):
            return pltpu.async_copy(h_hbm.at[src_a.at[pl.ds(win * B, B)]],
                                    rows_v[p], gsem[p])

        def wait_gather(p, win):
            pltpu.make_async_copy(h_hbm.at[src_a.at[pl.ds(win * B, B)]],
                                  rows_v[p], gsem[p]).wait()

        def issue_dstload(p, win):
            pltpu.async_copy(dst_hbm.at[pl.ds(tbase + win * B, B)],
                             dst_v[p], dsem[p])
            pltpu.async_copy(w_hbm.at[pl.ds(tbase + win * B, B)],
                             w_v[p], dsem[p])

        def wait_dstload(p, win):
            pltpu.make_async_copy(dst_hbm.at[pl.ds(tbase + win * B, B)],
                                  dst_v[p], dsem[p]).wait()
            pltpu.make_async_copy(w_hbm.at[pl.ds(tbase + win * B, B)],
                                  w_v[p], dsem[p]).wait()

        UNROLL = 1

        def scale(p, win):
            def grp(g, carry):
                for u in range(UNROLL):
                    gg = g * UNROLL + u
                    w16 = w_v[p][pl.ds(gg * LANES, LANES)]
                    for j in range(LANES):
                        we = w16[j]
                        e = gg * LANES + j
                        for kk in range(KD):
                            sl = pl.ds(kk * LANES, LANES)
                            rows_v[p][e, sl] = rows_v[p][e, sl] * we
                return carry

            lax.fori_loop(0, B // LANES // UNROLL, grp, 0)

        def issue_scatter(p, win):
            return pltpu.async_copy(rows_v[p], acc.at[dst_v[p]],
                                    ssem[p], add=True)

        def wait_scatter(p, win):
            pltpu.make_async_copy(rows_v[p], acc.at[dst_v[p]],
                                  ssem[p]).wait()

        issue_dstload(0, 0)
        issue_gather(0, 0)

        def step(win, p, first, last):
            wait_gather(p, win)
            if not last:
                if not first:
                    wait_scatter(1 - p, win - 1)
                issue_gather(1 - p, win + 1)
                issue_dstload(1 - p, win + 1)
            scale(p, win)
            wait_dstload(p, win)
            issue_scatter(p, win)
            if last:
                wait_scatter(1 - p, win - 1)
                wait_scatter(p, win)

        step(0, 0, True, False)
        npairs = (NWIN - 2) // 2

        def pair(q, carry):
            win = 1 + 2 * q
            step(win, 1, False, False)
            step(win + 1, 0, False, False)
            return carry

        lax.fori_loop(0, npairs, pair, 0)
        if NWIN - 1 - 2 * npairs == 2:
            step(NWIN - 2, (NWIN - 2) % 2, False, False)
        step(NWIN - 1, (NWIN - 1) % 2, False, True)
        plsc.subcore_barrier()
        for j in range((NRCH + NS - 1) // NS):
            ch = s + j * NS

            @pl.when(ch < NRCH)
            def _():
                pltpu.sync_copy(acc.at[pl.ds(ch * RCH, RCH)],
                                rows_v[0].at[pl.ds(0, RCH)])
                pltpu.sync_copy(rows_v[0].at[pl.ds(0, RCH)],
                                out_hbm.at[c, pl.ds(ch * RCH, RCH)])

    return k(hp, src, dst, w)


_RB = 2000


def _first_layer_h(x, W, degp):
    N, D = x.shape

    def body(x_ref, w_ref, dp_ref, o_ref, d_ref):
        deg = dp_ref[:, 0] + dp_ref[:, 1] + 1.0
        dis = jnp.where(deg > 0, lax.rsqrt(deg), 0.0)[:, None]
        d_ref[...] = dis
        o_ref[...] = dis * jnp.dot(
            x_ref[...], w_ref[...], preferred_element_type=jnp.float32)

    return pl.pallas_call(
        body,
        grid=(N // _RB,),
        in_specs=[
            pl.BlockSpec((_RB, D), lambda i: (i, 0)),
            pl.BlockSpec((D, D), lambda i: (0, 0)),
            pl.BlockSpec((_RB, 2), lambda i: (i, 0)),
        ],
        out_specs=[
            pl.BlockSpec((_RB, D), lambda i: (i, 0)),
            pl.BlockSpec((_RB, 1), lambda i: (i, 0)),
        ],
        out_shape=[
            jax.ShapeDtypeStruct((N, D), jnp.float32),
            jax.ShapeDtypeStruct((N, 1), jnp.float32),
        ],
    )(x, W, degp)


def _mid_layer(accp, hp, dis, b, W):
    N, D = hp.shape

    def body(a_ref, h_ref, d_ref, b_ref, w_ref, o_ref):
        ssum = a_ref[0] + a_ref[1] + h_ref[...]
        x2 = jnp.maximum(d_ref[...] * ssum + b_ref[...], 0.0)
        o_ref[...] = d_ref[...] * jnp.dot(
            x2, w_ref[...], preferred_element_type=jnp.float32)

    return pl.pallas_call(
        body,
        grid=(N // _RB,),
        in_specs=[
            pl.BlockSpec((NC, _RB, D), lambda i: (0, i, 0)),
            pl.BlockSpec((_RB, D), lambda i: (i, 0)),
            pl.BlockSpec((_RB, 1), lambda i: (i, 0)),
            pl.BlockSpec((1, D), lambda i: (0, 0)),
            pl.BlockSpec((D, D), lambda i: (0, 0)),
        ],
        out_specs=pl.BlockSpec((_RB, D), lambda i: (i, 0)),
        out_shape=jax.ShapeDtypeStruct((N, D), jnp.float32),
    )(accp, hp, dis, b, W)


def _final_layer(accp, hp, dis, b, Wl, bl):
    N, D = hp.shape

    def body(a_ref, h_ref, d_ref, b_ref, wl_ref, bl_ref, o_ref):
        ssum = a_ref[0] + a_ref[1] + h_ref[...]
        x3 = jnp.maximum(d_ref[...] * ssum + b_ref[...], 0.0)
        o_ref[...] = jnp.dot(
            x3, wl_ref[...], preferred_element_type=jnp.float32) + bl_ref[...]

    return pl.pallas_call(
        body,
        grid=(N // _RB,),
        in_specs=[
            pl.BlockSpec((NC, _RB, D), lambda i: (0, i, 0)),
            pl.BlockSpec((_RB, D), lambda i: (i, 0)),
            pl.BlockSpec((_RB, 1), lambda i: (i, 0)),
            pl.BlockSpec((1, D), lambda i: (0, 0)),
            pl.BlockSpec((D, 1), lambda i: (0, 0)),
            pl.BlockSpec((1, 1), lambda i: (0, 0)),
        ],
        out_specs=pl.BlockSpec((_RB, 1), lambda i: (i, 0)),
        out_shape=jax.ShapeDtypeStruct((N, 1), jnp.float32),
    )(accp, hp, dis, b, Wl, bl)


def kernel(w_x, edge_index, edge_weight, W1, b1, W2, b2, Wl, bl):
    N = w_x.shape[0]
    src = edge_index[0].astype(jnp.int32)
    dst = edge_index[1].astype(jnp.int32)
    w = edge_weight.astype(jnp.float32)

    E = src.shape[0]
    EP = ((E // NW + 127) // 128) * 128 * NW
    npad = EP - E
    if npad:
        pad_idx = (jnp.arange(npad, dtype=jnp.int32) * 97) % N
        src_p = jnp.concatenate([src, pad_idx])
        dst_p = jnp.concatenate([dst, pad_idx])
        w_p = jnp.concatenate([w, jnp.zeros((npad,), jnp.float32)])
    else:
        src_p, dst_p, w_p = src, dst, w

    degp = _deg_partials(dst, w, N)
    h1p, dis = _first_layer_h(w_x, W1, degp.T)
    acc1 = _aggregate(h1p, src_p, dst_p, w_p)
    h2p = _mid_layer(acc1, h1p, dis, b1.reshape(1, -1), W2)
    acc2 = _aggregate(h2p, src_p, dst_p, w_p)
    return _final_layer(acc2, h2p, dis, b2.reshape(1, -1), Wl,
                        bl.reshape(1, 1))


# speedup vs baseline: 1.0240x; 1.0058x over previous
import functools

import jax
import jax.numpy as jnp
from jax import lax
from jax.experimental import pallas as pl
from jax.experimental.pallas import tpu as pltpu
from jax.experimental.pallas import tpu_sc as plsc

NC, NS = 2, 16
NW = NC * NS
LANES = 16


def _deg_partials(dst, w, N):
    E = dst.shape[0]
    EW = E // NW
    CH = 400
    NCH = N // CH
    mesh = plsc.VectorSubcoreMesh(core_axis_name="c", subcore_axis_name="s",
                                  num_cores=NC, num_subcores=NS)

    @functools.partial(
        pl.kernel,
        out_type=jax.ShapeDtypeStruct((NC * N,), jnp.float32),
        mesh=mesh,
        scratch_types=[
            pltpu.VMEM((EW,), jnp.int32),
            pltpu.VMEM((EW,), jnp.float32),
            pltpu.VMEM((CH,), jnp.float32),
            pltpu.VMEM_SHARED((N,), jnp.float32),
        ],
    )
    def k(dst_hbm, w_hbm, out_hbm, dst_v, w_v, zbuf, deg_sp):
        c = lax.axis_index("c")
        s = lax.axis_index("s")
        wid = c * NS + s

        def zb(i, carry):
            zbuf[pl.ds(i * LANES, LANES)] = jnp.zeros((LANES,), jnp.float32)
            return carry

        lax.fori_loop(0, CH // LANES, zb, 0)
        for j in range((NCH + NS - 1) // NS):
            ch = s + j * NS

            @pl.when(ch < NCH)
            def _():
                pltpu.sync_copy(zbuf, deg_sp.at[pl.ds(ch * CH, CH)])

        plsc.subcore_barrier()
        base = wid * EW
        pltpu.sync_copy(dst_hbm.at[pl.ds(base, EW)], dst_v)
        pltpu.sync_copy(w_hbm.at[pl.ds(base, EW)], w_v)
        pltpu.sync_copy(w_v, deg_sp.at[dst_v], add=True)
        plsc.subcore_barrier()
        for j in range((NCH + NS - 1) // NS):
            ch = s + j * NS

            @pl.when(ch < NCH)
            def _():
                pltpu.sync_copy(deg_sp.at[pl.ds(ch * CH, CH)], zbuf)
                pltpu.sync_copy(zbuf, out_hbm.at[pl.ds(c * N + ch * CH, CH)])

    return k(dst, w).reshape(NC, N)


def _aggregate(hp, src, dst, w):
    N, D = hp.shape
    E = src.shape[0]
    EW = E // NW
    B = 128
    NWIN = EW // B
    RCH = 80
    NRCH = N // RCH
    KD = D // LANES
    mesh = plsc.VectorSubcoreMesh(core_axis_name="c", subcore_axis_name="s",
                                  num_cores=NC, num_subcores=NS)

    @functools.partial(
        pl.kernel,
        out_type=jax.ShapeDtypeStruct((NC, N, D), jnp.float32),
        mesh=mesh,
        scratch_types=[
            pltpu.VMEM((EW,), jnp.int32),
            [pltpu.VMEM((B,), jnp.int32) for _ in range(2)],
            [pltpu.VMEM((B,), jnp.float32) for _ in range(2)],
            [pltpu.VMEM((B, D), jnp.float32) for _ in range(2)],
            pltpu.VMEM_SHARED((N, D), jnp.float32),
            [pltpu.SemaphoreType.DMA for _ in range(2)],
            [pltpu.SemaphoreType.DMA for _ in range(2)],
            [pltpu.SemaphoreType.DMA for _ in range(2)],
        ],
    )
    def k(h_hbm, src_hbm, dst_hbm, w_hbm, out_hbm,
          src_a, dst_v, w_v, rows_v, acc, gsem, ssem, dsem):
        c = lax.axis_index("c")
        s = lax.axis_index("s")
        wid = c * NS + s
        tbase = wid * EW

        pltpu.sync_copy(src_hbm.at[pl.ds(tbase, EW)], src_a)

        def zr(e, carry):
            for kk in range(KD):
                rows_v[0][e, pl.ds(kk * LANES, LANES)] = jnp.zeros(
                    (LANES,), jnp.float32)
            return carry

        lax.fori_loop(0, RCH, zr, 0)
        for j in range((NRCH + NS - 1) // NS):
            ch = s + j * NS

            @pl.when(ch < NRCH)
            def _():
                pltpu.sync_copy(rows_v[0].at[pl.ds(0, RCH)],
                                acc.at[pl.ds(ch * RCH, RCH)])

        plsc.subcore_barrier()

        def issue_gather(p, win):
            return pltpu.async_copy(h_hbm.at[src_a.at[pl.ds(win * B, B)]],
                                    rows_v[p], gsem[p])

        def wait_gather(p, win):
            pltpu.make_async_copy(h_hbm.at[src_a.at[pl.ds(win * B, B)]],
                                  rows_v[p], gsem[p]).wait()

        def issue_dstload(p, win):
            pltpu.async_copy(dst_hbm.at[pl.ds(tbase + win * B, B)],
                             dst_v[p], dsem[p])
            pltpu.async_copy(w_hbm.at[pl.ds(tbase + win * B, B)],
                             w_v[p], dsem[p])

        def wait_dstload(p, win):
            pltpu.make_async_copy(dst_hbm.at[pl.ds(tbase + win * B, B)],
                                  dst_v[p], dsem[p]).wait()
            pltpu.make_async_copy(w_hbm.at[pl.ds(tbase + win * B, B)],
                                  w_v[p], dsem[p]).wait()

        UNROLL = 1

        def scale(p, win):
            def grp(g, carry):
                for u in range(UNROLL):
                    gg = g * UNROLL + u
                    w16 = w_v[p][pl.ds(gg * LANES, LANES)]
                    for j in range(LANES):
                        we = w16[j]
                        e = gg * LANES + j
                        for kk in range(KD):
                            sl = pl.ds(kk * LANES, LANES)
                            rows_v[p][e, sl] = rows_v[p][e, sl] * we
                return carry

            lax.fori_loop(0, B // LANES // UNROLL, grp, 0)

        def issue_scatter(p, win):
            return pltpu.async_copy(rows_v[p], acc.at[dst_v[p]],
                                    ssem[p], add=True)

        def wait_scatter(p, win):
            pltpu.make_async_copy(rows_v[p], acc.at[dst_v[p]],
                                  ssem[p]).wait()

        issue_dstload(0, 0)
        issue_gather(0, 0)

        def step(win, p, first, last):
            wait_gather(p, win)
            if not last:
                if not first:
                    wait_scatter(1 - p, win - 1)
                issue_gather(1 - p, win + 1)
                issue_dstload(1 - p, win + 1)
            wait_dstload(p, win)
            scale(p, win)
            issue_scatter(p, win)
            if last:
                wait_scatter(1 - p, win - 1)
                wait_scatter(p, win)

        step(0, 0, True, False)
        npairs = (NWIN - 2) // 2

        def pair(q, carry):
            win = 1 + 2 * q
            step(win, 1, False, False)
            step(win + 1, 0, False, False)
            return carry

        lax.fori_loop(0, npairs, pair, 0)
        if NWIN - 1 - 2 * npairs == 2:
            step(NWIN - 2, (NWIN - 2) % 2, False, False)
        step(NWIN - 1, (NWIN - 1) % 2, False, True)
        plsc.subcore_barrier()
        for j in range((NRCH + NS - 1) // NS):
            ch = s + j * NS

            @pl.when(ch < NRCH)
            def _():
                pltpu.sync_copy(acc.at[pl.ds(ch * RCH, RCH)],
                                rows_v[0].at[pl.ds(0, RCH)])
                pltpu.sync_copy(rows_v[0].at[pl.ds(0, RCH)],
                                out_hbm.at[c, pl.ds(ch * RCH, RCH)])

    return k(hp, src, dst, w)


_RB = 2000


def _first_layer_h(x, W, degp):
    N, D = x.shape

    def body(x_ref, w_ref, dp_ref, o_ref, d_ref):
        deg = dp_ref[:, 0] + dp_ref[:, 1] + 1.0
        dis = jnp.where(deg > 0, lax.rsqrt(deg), 0.0)[:, None]
        d_ref[...] = dis
        o_ref[...] = dis * jnp.dot(
            x_ref[...], w_ref[...], preferred_element_type=jnp.float32)

    return pl.pallas_call(
        body,
        grid=(N // _RB,),
        in_specs=[
            pl.BlockSpec((_RB, D), lambda i: (i, 0)),
            pl.BlockSpec((D, D), lambda i: (0, 0)),
            pl.BlockSpec((_RB, 2), lambda i: (i, 0)),
        ],
        out_specs=[
            pl.BlockSpec((_RB, D), lambda i: (i, 0)),
            pl.BlockSpec((_RB, 1), lambda i: (i, 0)),
        ],
        out_shape=[
            jax.ShapeDtypeStruct((N, D), jnp.float32),
            jax.ShapeDtypeStruct((N, 1), jnp.float32),
        ],
    )(x, W, degp)


def _mid_layer(accp, hp, dis, b, W):
    N, D = hp.shape

    def body(a_ref, h_ref, d_ref, b_ref, w_ref, o_ref):
        ssum = a_ref[0] + a_ref[1] + h_ref[...]
        x2 = jnp.maximum(d_ref[...] * ssum + b_ref[...], 0.0)
        o_ref[...] = d_ref[...] * jnp.dot(
            x2, w_ref[...], preferred_element_type=jnp.float32)

    return pl.pallas_call(
        body,
        grid=(N // _RB,),
        in_specs=[
            pl.BlockSpec((NC, _RB, D), lambda i: (0, i, 0)),
            pl.BlockSpec((_RB, D), lambda i: (i, 0)),
            pl.BlockSpec((_RB, 1), lambda i: (i, 0)),
            pl.BlockSpec((1, D), lambda i: (0, 0)),
            pl.BlockSpec((D, D), lambda i: (0, 0)),
        ],
        out_specs=pl.BlockSpec((_RB, D), lambda i: (i, 0)),
        out_shape=jax.ShapeDtypeStruct((N, D), jnp.float32),
    )(accp, hp, dis, b, W)


def _final_layer(accp, hp, dis, b, Wl, bl):
    N, D = hp.shape

    def body(a_ref, h_ref, d_ref, b_ref, wl_ref, bl_ref, o_ref):
        ssum = a_ref[0] + a_ref[1] + h_ref[...]
        x3 = jnp.maximum(d_ref[...] * ssum + b_ref[...], 0.0)
        o_ref[...] = jnp.dot(
            x3, wl_ref[...], preferred_element_type=jnp.float32) + bl_ref[...]

    return pl.pallas_call(
        body,
        grid=(N // _RB,),
        in_specs=[
            pl.BlockSpec((NC, _RB, D), lambda i: (0, i, 0)),
            pl.BlockSpec((_RB, D), lambda i: (i, 0)),
            pl.BlockSpec((_RB, 1), lambda i: (i, 0)),
            pl.BlockSpec((1, D), lambda i: (0, 0)),
            pl.BlockSpec((D, 1), lambda i: (0, 0)),
            pl.BlockSpec((1, 1), lambda i: (0, 0)),
        ],
        out_specs=pl.BlockSpec((_RB, 1), lambda i: (i, 0)),
        out_shape=jax.ShapeDtypeStruct((N, 1), jnp.float32),
    )(accp, hp, dis, b, Wl, bl)


def kernel(w_x, edge_index, edge_weight, W1, b1, W2, b2, Wl, bl):
    N = w_x.shape[0]
    src = edge_index[0].astype(jnp.int32)
    dst = edge_index[1].astype(jnp.int32)
    w = edge_weight.astype(jnp.float32)

    E = src.shape[0]
    EP = ((E // NW + 127) // 128) * 128 * NW
    npad = EP - E
    if npad:
        pad_idx = (jnp.arange(npad, dtype=jnp.int32) * 97) % N
        src_p = jnp.concatenate([src, pad_idx])
        dst_p = jnp.concatenate([dst, pad_idx])
        w_p = jnp.concatenate([w, jnp.zeros((npad,), jnp.float32)])
    else:
        src_p, dst_p, w_p = src, dst, w

    degp = _deg_partials(dst, w, N)
    h1p, dis = _first_layer_h(w_x, W1, degp.T)
    acc1 = _aggregate(h1p, src_p, dst_p, w_p)
    h2p = _mid_layer(acc1, h1p, dis, b1.reshape(1, -1), W2)
    acc2 = _aggregate(h2p, src_p, dst_p, w_p)
    return _final_layer(acc2, h2p, dis, b2.reshape(1, -1), Wl,
                        bl.reshape(1, 1))
